# baseline (speedup 1.0000x reference)
import numpy as np

import jax
import jax.numpy as jnp
from jax.experimental import pallas as pl
from jax.experimental.pallas import tpu as pltpu

K = 5
C_IN, H_IN, W_IN = 3, 32, 32
OC1, OC2 = 6, 16
B_TOTAL = 2048

M1 = OC1 * 28
M1P = 176
M2 = OC2 * 10
KROWS1 = C_IN * H_IN
OUT_ROWS = 16

G_IMGS = 256
NL = G_IMGS * W_IN


def _rotl(a, k):
    if k == 0:
        return a
    return jnp.concatenate([a[:, k:], a[:, :k]], axis=1)


def _rotu(a):
    return jnp.concatenate([a[1:], a[:1]], axis=0)


def _lenet_kernel(x_ref, w1_ref, b1_ref, w2_ref, b2_ref, wf1_ref, bf1_ref,
                  wf2_ref, bf2_ref, wf3_ref, bf3_ref, o_ref):
    x = x_ref[...]

    acc1 = jnp.zeros((M1P, NL), jnp.float32)
    for kw in range(K):
        acc1 = acc1 + jnp.dot(w1_ref[kw], _rotl(x, kw),
                              preferred_element_type=jnp.float32)
    r1 = jnp.maximum(acc1 + b1_ref[...], 0.0)

    m1 = jnp.maximum(r1, _rotu(r1))
    p1 = jnp.maximum(m1, _rotl(m1, 1)).astype(jnp.bfloat16)

    acc2 = jnp.zeros((M2, NL), jnp.float32)
    for kw in range(K):
        acc2 = acc2 + jnp.dot(w2_ref[kw], _rotl(p1, 2 * kw),
                              preferred_element_type=jnp.float32)
    r2 = jnp.maximum(acc2 + b2_ref[...], 0.0)

    m2 = jnp.maximum(r2, _rotu(r2))
    p2 = jnp.maximum(m2, _rotl(m2, 2)).astype(jnp.bfloat16)

    h1 = jnp.zeros((128, NL), jnp.float32)
    for fj in range(K):
        h1 = h1 + jnp.dot(wf1_ref[fj], _rotl(p2, 4 * fj),
                          preferred_element_type=jnp.float32)
    h1 = jnp.maximum(h1 + bf1_ref[...], 0.0).astype(jnp.bfloat16)

    h2 = jnp.maximum(jnp.dot(wf2_ref[...], h1,
                             preferred_element_type=jnp.float32)
                     + bf2_ref[...], 0.0).astype(jnp.bfloat16)
    o_ref[...] = (jnp.dot(wf3_ref[...], h2,
                          preferred_element_type=jnp.float32)
                  + bf3_ref[...])


def _const_diag1():
    d = np.zeros((K, 28, H_IN), np.float32)
    for kh in range(K):
        for i in range(28):
            d[kh, i, i + kh] = 1.0
    return jnp.asarray(d)


def _const_diag2():
    d = np.zeros((K, 10, 28), np.float32)
    for kh in range(K):
        for io in range(10):
            d[kh, io, 2 * (io + kh)] = 1.0
    return jnp.asarray(d)


@jax.jit
def _lenet_fwd(conv1_w, conv1_b, conv2_w, conv2_b, fc1_w, fc1_b,
               fc2_w, fc2_b, fc3_w, fc3_b, x):
    B = x.shape[0]

    w1r = conv1_w.reshape(K, K, 8, 8)[:, :, :OC1, :C_IN]
    w1t = jnp.einsum('aih,awoc->woich', _const_diag1(), w1r)
    w1t = jnp.pad(w1t.reshape(K, M1, KROWS1), ((0, 0), (0, M1P - M1), (0, 0)))
    b1c = jnp.pad(jnp.repeat(conv1_b[:OC1, 0], 28), (0, M1P - M1))
    b1c = b1c.reshape(M1P, 1)

    w2r = conv2_w.reshape(K, K, 16, 8)[:, :, :, :OC1]
    w2t = jnp.einsum('aih,awoc->woich', _const_diag2(), w2r)
    w2t = jnp.pad(w2t.reshape(K, M2, OC1 * 28),
                  ((0, 0), (0, 0), (0, M1P - M1)))
    b2c = jnp.repeat(conv2_b[:, 0], 10).reshape(M2, 1)

    wf = fc1_w[:OC2 * 25].reshape(OC2, K, K, 128)
    wf = wf.transpose(2, 3, 0, 1)
    wf1 = jnp.stack([wf, jnp.zeros_like(wf)], axis=-1)
    wf1 = wf1.reshape(K, 128, OC2, 10).reshape(K, 128, M2)
    bf1 = fc1_b.reshape(128, 1)

    wf2 = fc2_w.T
    bf2 = fc2_b.reshape(128, 1)
    wf3 = fc3_w.T[:OUT_ROWS]
    bf3 = fc3_b[0, :OUT_ROWS].reshape(OUT_ROWS, 1)

    xt = x.transpose(1, 2, 0, 3).reshape(KROWS1, B * W_IN)
    xt = xt.astype(jnp.bfloat16)

    bf = jnp.bfloat16
    w1t, w2t, wf1, wf2, wf3 = (a.astype(bf) for a in (w1t, w2t, wf1, wf2, wf3))

    grid = (B * W_IN // NL,)
    out = pl.pallas_call(
        _lenet_kernel,
        out_shape=jax.ShapeDtypeStruct((OUT_ROWS, B * W_IN), jnp.float32),
        grid=grid,
        in_specs=[
            pl.BlockSpec((KROWS1, NL), lambda i: (0, i)),
            pl.BlockSpec((K, M1P, KROWS1), lambda i: (0, 0, 0)),
            pl.BlockSpec((M1P, 1), lambda i: (0, 0)),
            pl.BlockSpec((K, M2, M1P), lambda i: (0, 0, 0)),
            pl.BlockSpec((M2, 1), lambda i: (0, 0)),
            pl.BlockSpec((K, 128, M2), lambda i: (0, 0, 0)),
            pl.BlockSpec((128, 1), lambda i: (0, 0)),
            pl.BlockSpec((128, 128), lambda i: (0, 0)),
            pl.BlockSpec((128, 1), lambda i: (0, 0)),
            pl.BlockSpec((OUT_ROWS, 128), lambda i: (0, 0)),
            pl.BlockSpec((OUT_ROWS, 1), lambda i: (0, 0)),
        ],
        out_specs=pl.BlockSpec((OUT_ROWS, NL), lambda i: (0, i)),
        compiler_params=pltpu.CompilerParams(
            dimension_semantics=("parallel",)),
    )(xt, w1t, b1c, w2t, b2c, wf1, bf1, wf2, bf2, wf3, bf3)

    return out[:10, ::W_IN].T


def kernel(conv1_w, conv1_b, conv2_w, conv2_b, fc1_w, fc1_b,
           fc2_w, fc2_b, fc3_w, fc3_b, x):
    return _lenet_fwd(conv1_w, conv1_b, conv2_w, conv2_b, fc1_w, fc1_b,
                      fc2_w, fc2_b, fc3_w, fc3_b, x)


# speedup vs baseline: 1.0175x; 1.0175x over previous
import numpy as np

import jax
import jax.numpy as jnp
from jax.experimental import pallas as pl
from jax.experimental.pallas import tpu as pltpu

K = 5
C_IN, H_IN, W_IN = 3, 32, 32
OC1, OC2 = 6, 16
B_TOTAL = 2048

M1 = OC1 * 28
M1P = 176
M2 = OC2 * 10
KROWS1 = C_IN * H_IN
OUT_ROWS = 16

G_IMGS = 128
NL = G_IMGS * W_IN


def _rotl(a, k):
    if k == 0:
        return a
    return jnp.concatenate([a[:, k:], a[:, :k]], axis=1)


def _rotu(a):
    return jnp.concatenate([a[1:], a[:1]], axis=0)


def _lenet_kernel(x_ref, w1_ref, b1_ref, w2_ref, b2_ref, wf1_ref, bf1_ref,
                  wf2_ref, bf2_ref, wf3_ref, bf3_ref, o_ref):
    h = NL // 2
    _half_net(x_ref[:, :h], w1_ref, b1_ref, w2_ref, b2_ref, wf1_ref, bf1_ref,
              wf2_ref, bf2_ref, wf3_ref, bf3_ref, o_ref.at[:, :h])
    _half_net(x_ref[:, h:], w1_ref, b1_ref, w2_ref, b2_ref, wf1_ref, bf1_ref,
              wf2_ref, bf2_ref, wf3_ref, bf3_ref, o_ref.at[:, h:])


def _half_net(x, w1_ref, b1_ref, w2_ref, b2_ref, wf1_ref, bf1_ref,
              wf2_ref, bf2_ref, wf3_ref, bf3_ref, o_ref):

    acc1 = jnp.zeros((M1P, NL // 2), jnp.float32)
    for kw in range(K):
        acc1 = acc1 + jnp.dot(w1_ref[kw], _rotl(x, kw),
                              preferred_element_type=jnp.float32)
    r1 = jnp.maximum(acc1 + b1_ref[...], 0.0)

    m1 = jnp.maximum(r1, _rotu(r1))
    p1 = jnp.maximum(m1, _rotl(m1, 1)).astype(jnp.bfloat16)

    acc2 = jnp.zeros((M2, NL // 2), jnp.float32)
    for kw in range(K):
        acc2 = acc2 + jnp.dot(w2_ref[kw], _rotl(p1, 2 * kw),
                              preferred_element_type=jnp.float32)
    r2 = jnp.maximum(acc2 + b2_ref[...], 0.0)

    m2 = jnp.maximum(r2, _rotu(r2))
    p2 = jnp.maximum(m2, _rotl(m2, 2)).astype(jnp.bfloat16)

    h1 = jnp.zeros((128, NL // 2), jnp.float32)
    for fj in range(K):
        h1 = h1 + jnp.dot(wf1_ref[fj], _rotl(p2, 4 * fj),
                          preferred_element_type=jnp.float32)
    h1 = jnp.maximum(h1 + bf1_ref[...], 0.0).astype(jnp.bfloat16)

    h2 = jnp.maximum(jnp.dot(wf2_ref[...], h1,
                             preferred_element_type=jnp.float32)
                     + bf2_ref[...], 0.0).astype(jnp.bfloat16)
    o_ref[...] = (jnp.dot(wf3_ref[...], h2,
                          preferred_element_type=jnp.float32)
                  + bf3_ref[...])
    return None


def _const_diag1():
    d = np.zeros((K, 28, H_IN), np.float32)
    for kh in range(K):
        for i in range(28):
            d[kh, i, i + kh] = 1.0
    return jnp.asarray(d)


def _const_diag2():
    d = np.zeros((K, 10, 28), np.float32)
    for kh in range(K):
        for io in range(10):
            d[kh, io, 2 * (io + kh)] = 1.0
    return jnp.asarray(d)


@jax.jit
def _lenet_fwd(conv1_w, conv1_b, conv2_w, conv2_b, fc1_w, fc1_b,
               fc2_w, fc2_b, fc3_w, fc3_b, x):
    B = x.shape[0]

    w1r = conv1_w.reshape(K, K, 8, 8)[:, :, :OC1, :C_IN]
    w1t = jnp.einsum('aih,awoc->woich', _const_diag1(), w1r)
    w1t = jnp.pad(w1t.reshape(K, M1, KROWS1), ((0, 0), (0, M1P - M1), (0, 0)))
    b1c = jnp.pad(jnp.repeat(conv1_b[:OC1, 0], 28), (0, M1P - M1))
    b1c = b1c.reshape(M1P, 1)

    w2r = conv2_w.reshape(K, K, 16, 8)[:, :, :, :OC1]
    w2t = jnp.einsum('aih,awoc->woich', _const_diag2(), w2r)
    w2t = jnp.pad(w2t.reshape(K, M2, OC1 * 28),
                  ((0, 0), (0, 0), (0, M1P - M1)))
    b2c = jnp.repeat(conv2_b[:, 0], 10).reshape(M2, 1)

    wf = fc1_w[:OC2 * 25].reshape(OC2, K, K, 128)
    wf = wf.transpose(2, 3, 0, 1)
    wf1 = jnp.stack([wf, jnp.zeros_like(wf)], axis=-1)
    wf1 = wf1.reshape(K, 128, OC2, 10).reshape(K, 128, M2)
    bf1 = fc1_b.reshape(128, 1)

    wf2 = fc2_w.T
    bf2 = fc2_b.reshape(128, 1)
    wf3 = fc3_w.T[:OUT_ROWS]
    bf3 = fc3_b[0, :OUT_ROWS].reshape(OUT_ROWS, 1)

    xt = x.transpose(1, 2, 0, 3).reshape(KROWS1, B * W_IN)
    xt = xt.astype(jnp.bfloat16)

    bf = jnp.bfloat16
    w1t, w2t, wf1, wf2, wf3 = (a.astype(bf) for a in (w1t, w2t, wf1, wf2, wf3))

    grid = (B * W_IN // NL,)
    out = pl.pallas_call(
        _lenet_kernel,
        out_shape=jax.ShapeDtypeStruct((OUT_ROWS, B * W_IN), jnp.float32),
        grid=grid,
        in_specs=[
            pl.BlockSpec((KROWS1, NL), lambda i: (0, i)),
            pl.BlockSpec((K, M1P, KROWS1), lambda i: (0, 0, 0)),
            pl.BlockSpec((M1P, 1), lambda i: (0, 0)),
            pl.BlockSpec((K, M2, M1P), lambda i: (0, 0, 0)),
            pl.BlockSpec((M2, 1), lambda i: (0, 0)),
            pl.BlockSpec((K, 128, M2), lambda i: (0, 0, 0)),
            pl.BlockSpec((128, 1), lambda i: (0, 0)),
            pl.BlockSpec((128, 128), lambda i: (0, 0)),
            pl.BlockSpec((128, 1), lambda i: (0, 0)),
            pl.BlockSpec((OUT_ROWS, 128), lambda i: (0, 0)),
            pl.BlockSpec((OUT_ROWS, 1), lambda i: (0, 0)),
        ],
        out_specs=pl.BlockSpec((OUT_ROWS, NL), lambda i: (0, i)),
        compiler_params=pltpu.CompilerParams(
            dimension_semantics=("parallel",)),
    )(xt, w1t, b1c, w2t, b2c, wf1, bf1, wf2, bf2, wf3, bf3)

    return out[:10, ::W_IN].T


def kernel(conv1_w, conv1_b, conv2_w, conv2_b, fc1_w, fc1_b,
           fc2_w, fc2_b, fc3_w, fc3_b, x):
    return _lenet_fwd(conv1_w, conv1_b, conv2_w, conv2_b, fc1_w, fc1_b,
                      fc2_w, fc2_b, fc3_w, fc3_b, x)


# speedup vs baseline: 1.0253x; 1.0077x over previous
import numpy as np

import jax
import jax.numpy as jnp
from jax.experimental import pallas as pl
from jax.experimental.pallas import tpu as pltpu

K = 5
C_IN, H_IN, W_IN = 3, 32, 32
OC1, OC2 = 6, 16
B_TOTAL = 2048

M1 = OC1 * 28
M1P = 176
M2 = OC2 * 10
KROWS1 = C_IN * H_IN
OUT_ROWS = 16

G_IMGS = 128
NL = G_IMGS * W_IN


def _rotl(a, k):
    if k == 0:
        return a
    return jnp.concatenate([a[:, k:], a[:, :k]], axis=1)


def _rotu(a):
    return jnp.concatenate([a[1:], a[:1]], axis=0)


def _lenet_kernel(x_ref, w1_ref, b1_ref, w2_ref, b2_ref, wf1_ref, bf1_ref,
                  wf2_ref, bf2_ref, wf3_ref, bf3_ref, o_ref):
    x = x_ref[...]

    acc1 = jnp.zeros((M1P, NL), jnp.float32)
    for kw in range(K):
        acc1 = acc1 + jnp.dot(w1_ref[kw], _rotl(x, kw),
                              preferred_element_type=jnp.float32)
    r1 = jnp.maximum(acc1 + b1_ref[...], 0.0)

    m1 = jnp.maximum(r1, _rotu(r1))
    p1 = jnp.maximum(m1, _rotl(m1, 1)).astype(jnp.bfloat16)

    acc2 = jnp.zeros((M2, NL), jnp.float32)
    for kw in range(K):
        acc2 = acc2 + jnp.dot(w2_ref[kw], _rotl(p1, 2 * kw),
                              preferred_element_type=jnp.float32)
    r2 = jnp.maximum(acc2 + b2_ref[...], 0.0)

    m2 = jnp.maximum(r2, _rotu(r2))
    p2 = jnp.maximum(m2, _rotl(m2, 2)).astype(jnp.bfloat16)

    h1 = jnp.zeros((128, NL), jnp.float32)
    for fj in range(K):
        h1 = h1 + jnp.dot(wf1_ref[fj], _rotl(p2, 4 * fj),
                          preferred_element_type=jnp.float32)
    h1 = jnp.maximum(h1 + bf1_ref[...], 0.0).astype(jnp.bfloat16)

    h2 = jnp.maximum(jnp.dot(wf2_ref[...], h1,
                             preferred_element_type=jnp.float32)
                     + bf2_ref[...], 0.0).astype(jnp.bfloat16)
    o_ref[...] = (jnp.dot(wf3_ref[...], h2,
                          preferred_element_type=jnp.float32)
                  + bf3_ref[...])


def _const_diag1():
    d = np.zeros((K, 28, H_IN), np.float32)
    for kh in range(K):
        for i in range(28):
            d[kh, i, i + kh] = 1.0
    return jnp.asarray(d)


def _const_diag2():
    d = np.zeros((K, 10, 28), np.float32)
    for kh in range(K):
        for io in range(10):
            d[kh, io, 2 * (io + kh)] = 1.0
    return jnp.asarray(d)


@jax.jit
def _lenet_fwd(conv1_w, conv1_b, conv2_w, conv2_b, fc1_w, fc1_b,
               fc2_w, fc2_b, fc3_w, fc3_b, x):
    B = x.shape[0]

    w1r = conv1_w.reshape(K, K, 8, 8)[:, :, :OC1, :C_IN]
    w1t = jnp.einsum('aih,awoc->woich', _const_diag1(), w1r)
    w1t = jnp.pad(w1t.reshape(K, M1, KROWS1), ((0, 0), (0, M1P - M1), (0, 0)))
    b1c = jnp.pad(jnp.repeat(conv1_b[:OC1, 0], 28), (0, M1P - M1))
    b1c = b1c.reshape(M1P, 1)

    w2r = conv2_w.reshape(K, K, 16, 8)[:, :, :, :OC1]
    w2t = jnp.einsum('aih,awoc->woich', _const_diag2(), w2r)
    w2t = jnp.pad(w2t.reshape(K, M2, OC1 * 28),
                  ((0, 0), (0, 0), (0, M1P - M1)))
    b2c = jnp.repeat(conv2_b[:, 0], 10).reshape(M2, 1)

    wf = fc1_w[:OC2 * 25].reshape(OC2, K, K, 128)
    wf = wf.transpose(2, 3, 0, 1)
    wf1 = jnp.stack([wf, jnp.zeros_like(wf)], axis=-1)
    wf1 = wf1.reshape(K, 128, OC2, 10).reshape(K, 128, M2)
    bf1 = fc1_b.reshape(128, 1)

    wf2 = fc2_w.T
    bf2 = fc2_b.reshape(128, 1)
    wf3 = fc3_w.T[:OUT_ROWS]
    bf3 = fc3_b[0, :OUT_ROWS].reshape(OUT_ROWS, 1)

    xt = x.transpose(1, 2, 0, 3).reshape(KROWS1, B * W_IN)
    xt = xt.astype(jnp.bfloat16)

    bf = jnp.bfloat16
    w1t, w2t, wf1, wf2, wf3 = (a.astype(bf) for a in (w1t, w2t, wf1, wf2, wf3))

    grid = (B * W_IN // NL,)
    out = pl.pallas_call(
        _lenet_kernel,
        out_shape=jax.ShapeDtypeStruct((OUT_ROWS, B * W_IN), jnp.float32),
        grid=grid,
        in_specs=[
            pl.BlockSpec((KROWS1, NL), lambda i: (0, i)),
            pl.BlockSpec((K, M1P, KROWS1), lambda i: (0, 0, 0)),
            pl.BlockSpec((M1P, 1), lambda i: (0, 0)),
            pl.BlockSpec((K, M2, M1P), lambda i: (0, 0, 0)),
            pl.BlockSpec((M2, 1), lambda i: (0, 0)),
            pl.BlockSpec((K, 128, M2), lambda i: (0, 0, 0)),
            pl.BlockSpec((128, 1), lambda i: (0, 0)),
            pl.BlockSpec((128, 128), lambda i: (0, 0)),
            pl.BlockSpec((128, 1), lambda i: (0, 0)),
            pl.BlockSpec((OUT_ROWS, 128), lambda i: (0, 0)),
            pl.BlockSpec((OUT_ROWS, 1), lambda i: (0, 0)),
        ],
        out_specs=pl.BlockSpec((OUT_ROWS, NL), lambda i: (0, i)),
        compiler_params=pltpu.CompilerParams(
            dimension_semantics=("parallel",)),
    )(xt, w1t, b1c, w2t, b2c, wf1, bf1, wf2, bf2, wf3, bf3)

    return out[:10, ::W_IN].T


def kernel(conv1_w, conv1_b, conv2_w, conv2_b, fc1_w, fc1_b,
           fc2_w, fc2_b, fc3_w, fc3_b, x):
    return _lenet_fwd(conv1_w, conv1_b, conv2_w, conv2_b, fc1_w, fc1_b,
                      fc2_w, fc2_b, fc3_w, fc3_b, x)
